# baseline (speedup 1.0000x reference)
import functools

import jax
import jax.numpy as jnp
from jax import lax
from jax.experimental import pallas as pl
from jax.experimental.pallas import tpu as pltpu
from jax.experimental.pallas import tpu_sc as plsc

NC = 2
NS = 16
NW = NC * NS
LANES = 16


def kernel(x, table, weights, h):
    B = x.shape[0]
    R, D = table.shape
    V, C = h.shape
    BPW = B // NW
    CH = 64
    NCHUNK = BPW // CH
    NV = D // LANES

    mesh = plsc.VectorSubcoreMesh(
        core_axis_name="c", subcore_axis_name="s",
        num_cores=NC, num_subcores=NS)

    @functools.partial(
        pl.kernel,
        out_type=jax.ShapeDtypeStruct((B, D), jnp.float32),
        mesh=mesh,
        compiler_params=pltpu.CompilerParams(
            needs_layout_passes=False, use_tc_tiling_on_sc=False),
        scratch_types=[
            pltpu.VMEM((BPW,), jnp.int32),
            pltpu.VMEM((BPW * C,), jnp.int32),
            pltpu.VMEM((BPW * C,), jnp.int32),
            pltpu.VMEM((BPW * C,), jnp.float32),
            pltpu.VMEM((CH * C, D), jnp.float32),
            pltpu.VMEM((CH, D), jnp.float32),
            pltpu.SemaphoreType.DMA,
        ],
    )
    def sc_kernel(x_hbm, table_hbm, wflat_hbm, hflat_hbm, out_hbm,
                  x_v, idx_v, hx_v, w_v, rows_v, o_v, sem):
        wid = lax.axis_index("s") * NC + lax.axis_index("c")
        base = wid * BPW
        pltpu.sync_copy(x_hbm.at[pl.ds(base, BPW)], x_v)

        lanes = lax.iota(jnp.int32, LANES)

        def build(g, _):
            x16 = x_v[pl.ds(g * LANES, LANES)]
            pos0 = (g * LANES + lanes) * C
            val0 = x16 * C
            for c in range(C):
                plsc.store_scatter(idx_v, [pos0 + c], val0 + c)
            return ()

        lax.fori_loop(0, BPW // LANES, build, ())

        pltpu.async_copy(hflat_hbm.at[idx_v], hx_v, sem).wait()
        pltpu.async_copy(wflat_hbm.at[idx_v], w_v, sem).wait()

        for ch in range(NCHUNK):
            idx = hx_v.at[pl.ds(ch * CH * C, CH * C)]
            pltpu.async_copy(table_hbm.at[idx], rows_v, sem).wait()

            def body(b, _):
                acc = [None] * NV
                wbase = (ch * CH + b) * C
                for c in range(C):
                    ws = plsc.load_gather(
                        w_v, [jnp.full((LANES,), wbase + c, jnp.int32)])
                    for v in range(NV):
                        t = rows_v[b * C + c, pl.ds(v * LANES, LANES)] * ws
                        acc[v] = t if c == 0 else acc[v] + t
                for v in range(NV):
                    o_v[b, pl.ds(v * LANES, LANES)] = acc[v]
                return ()

            lax.fori_loop(0, CH, body, ())
            pltpu.sync_copy(o_v, out_hbm.at[pl.ds(base + ch * CH, CH)])

    return sc_kernel(x, table, weights.reshape(V * C), h.reshape(V * C))


# speedup vs baseline: 6.0901x; 6.0901x over previous
import functools

import jax
import jax.numpy as jnp
from jax import lax
from jax.experimental import pallas as pl
from jax.experimental.pallas import tpu as pltpu
from jax.experimental.pallas import tpu_sc as plsc

NC = 2
NS = 16
NW = NC * NS
LANES = 16


def kernel(x, table, weights, h):
    B = x.shape[0]
    R, D = table.shape
    V, C = h.shape
    BPW = B // NW
    CH = 64
    NCHUNK = BPW // CH
    NV = D // LANES

    mesh = plsc.VectorSubcoreMesh(
        core_axis_name="c", subcore_axis_name="s",
        num_cores=NC, num_subcores=NS)

    @functools.partial(
        pl.kernel,
        out_type=jax.ShapeDtypeStruct((B, D), jnp.float32),
        mesh=mesh,
        compiler_params=pltpu.CompilerParams(
            needs_layout_passes=False, use_tc_tiling_on_sc=False),
        scratch_types=[
            pltpu.VMEM((BPW * C,), jnp.int32),
            pltpu.VMEM((BPW * C,), jnp.float32),
            pltpu.VMEM((CH * C, D), jnp.float32),
            pltpu.VMEM((CH, D), jnp.float32),
            pltpu.SemaphoreType.DMA,
        ],
    )
    def sc_kernel(table_hbm, hxf_hbm, wxf_hbm, out_hbm,
                  hx_v, w_v, rows_v, o_v, sem):
        wid = lax.axis_index("s") * NC + lax.axis_index("c")
        base = wid * BPW
        pltpu.sync_copy(hxf_hbm.at[pl.ds(base * C, BPW * C)], hx_v)
        pltpu.sync_copy(wxf_hbm.at[pl.ds(base * C, BPW * C)], w_v)

        for ch in range(NCHUNK):
            idx = hx_v.at[pl.ds(ch * CH * C, CH * C)]
            pltpu.async_copy(table_hbm.at[idx], rows_v, sem).wait()

            def body(b, _):
                acc = [None] * NV
                wbase = (ch * CH + b) * C
                for c in range(C):
                    ws = plsc.load_gather(
                        w_v, [jnp.full((LANES,), wbase + c, jnp.int32)])
                    for v in range(NV):
                        t = rows_v[b * C + c, pl.ds(v * LANES, LANES)] * ws
                        acc[v] = t if c == 0 else acc[v] + t
                for v in range(NV):
                    o_v[b, pl.ds(v * LANES, LANES)] = acc[v]
                return ()

            lax.fori_loop(0, CH, body, ())
            pltpu.sync_copy(o_v, out_hbm.at[pl.ds(base + ch * CH, CH)])

    hxf = jnp.take(h, x, axis=0).reshape(B * C)
    wxf = jnp.take(weights, x, axis=0).reshape(B * C)
    return sc_kernel(table, hxf, wxf)


# speedup vs baseline: 7.1200x; 1.1691x over previous
import functools

import jax
import jax.numpy as jnp
from jax import lax
from jax.experimental import pallas as pl
from jax.experimental.pallas import tpu as pltpu
from jax.experimental.pallas import tpu_sc as plsc

NC = 2
NS = 16
NW = NC * NS
LANES = 16


def kernel(x, table, weights, h):
    B = x.shape[0]
    R, D = table.shape
    V, C = h.shape
    BPW = B // NW
    CH = 64
    NCHUNK = BPW // CH
    NV = D // LANES
    JB = BPW // 128
    DI = D // 8

    mesh = plsc.VectorSubcoreMesh(
        core_axis_name="c", subcore_axis_name="s",
        num_cores=NC, num_subcores=NS)

    @functools.partial(
        pl.kernel,
        out_type=jax.ShapeDtypeStruct((DI, B // 128, 1024), jnp.float32),
        mesh=mesh,
        compiler_params=pltpu.CompilerParams(
            needs_layout_passes=False, use_tc_tiling_on_sc=False),
        scratch_types=[
            pltpu.VMEM((JB, C, 128), jnp.int32),
            pltpu.VMEM((JB, C, 128), jnp.float32),
            pltpu.VMEM((BPW * C,), jnp.int32),
            pltpu.VMEM((BPW * C,), jnp.float32),
            pltpu.VMEM((2, CH * C, D), jnp.float32),
            pltpu.VMEM((DI * 1024,), jnp.float32),
            pltpu.SemaphoreType.DMA,
            pltpu.SemaphoreType.DMA,
        ],
    )
    def sc_kernel(table_hbm, hxl_hbm, wxl_hbm, out_hbm,
                  hxl_v, wxl_v, hx_v, w_v, rows_v, o_v, sem0, sem1):
        wid = lax.axis_index("s") * NC + lax.axis_index("c")
        base = wid * BPW
        pltpu.sync_copy(hxl_hbm.at[pl.ds(wid * JB, JB)], hxl_v)
        pltpu.sync_copy(wxl_hbm.at[pl.ds(wid * JB, JB)], wxl_v)

        lanes = lax.iota(jnp.int32, LANES)

        def flat(g, _):
            k = g * LANES + lanes
            b = k // C
            cc = k % C
            jb = b // 128
            l = b % 128
            hx_v[pl.ds(g * LANES, LANES)] = plsc.load_gather(
                hxl_v, [jb, cc, l])
            w_v[pl.ds(g * LANES, LANES)] = plsc.load_gather(
                wxl_v, [jb, cc, l])
            return ()

        lax.fori_loop(0, BPW * C // LANES, flat, ())

        sems = [sem0, sem1]
        posc = [((16 * v + jnp.arange(LANES)) // 8) * 1024
                + ((16 * v + jnp.arange(LANES)) % 8) * 128
                for v in range(NV)]

        def gather(ch):
            idx = hx_v.at[pl.ds(ch * CH * C, CH * C)]
            return pltpu.async_copy(
                table_hbm.at[idx], rows_v.at[ch % 2], sems[ch % 2])

        cp = gather(0)
        for jb in range(JB):
            for half in range(2):
                ch = jb * 2 + half
                nxt = gather(ch + 1) if ch + 1 < NCHUNK else None
                cp.wait()

                def body(b, _):
                    acc = [None] * NV
                    wbase = (ch * CH + b) * C
                    for c in range(C):
                        ws = plsc.load_gather(
                            w_v, [jnp.full((LANES,), wbase + c, jnp.int32)])
                        for v in range(NV):
                            t = rows_v[ch % 2, b * C + c,
                                       pl.ds(v * LANES, LANES)] * ws
                            acc[v] = t if c == 0 else acc[v] + t
                    l = half * CH + b
                    for v in range(NV):
                        pos = jnp.asarray(posc[v], jnp.int32) + l
                        plsc.store_scatter(o_v, [pos], acc[v])
                    return ()

                lax.fori_loop(0, CH, body, ())
                cp = nxt
            jg = wid * JB + jb
            for i in range(DI):
                pltpu.sync_copy(o_v.at[pl.ds(i * 1024, 1024)],
                                out_hbm.at[i, jg])

    hxl = jnp.take(h, x, axis=0).T.reshape(C, B // 128, 128).transpose(1, 0, 2)
    wxl = jnp.take(weights, x, axis=0).T.reshape(
        C, B // 128, 128).transpose(1, 0, 2)
    a = sc_kernel(table, hxl, wxl)
    return a.reshape(DI, B // 128, 8, 128).transpose(1, 3, 0, 2).reshape(B, D)
